# baseline (speedup 1.0000x reference)
import functools

import jax
import jax.numpy as jnp
from jax.experimental import pallas as pl
from jax.experimental.pallas import tpu as pltpu


def _crelu(x):
    return jnp.clip(x, 0.0, 1.0)


def _nnue_body(wf_ref, bf_ref, w8_ref, stm_ref, ftb8_ref, l1wT_ref, l1b_ref,
               l2wT_ref, l2b_ref, l3wT_ref, l3b_ref, out_ref,
               accA_ref, accC_ref):
    j = pl.program_id(1)

    w8 = w8_ref[...]
    pA = jnp.dot(wf_ref[...], w8, preferred_element_type=jnp.float32)
    pC = jnp.dot(bf_ref[...], w8, preferred_element_type=jnp.float32)

    @pl.when(j == 0)
    def _init():
        accA_ref[...] = pA
        accC_ref[...] = pC

    @pl.when(j > 0)
    def _acc():
        accA_ref[...] += pA
        accC_ref[...] += pC

    @pl.when(j == pl.num_programs(1) - 1)
    def _tail():
        A = accA_ref[...]
        C = accC_ref[...]
        lane = jax.lax.broadcasted_iota(jnp.int32, A.shape, 1)
        first_half = lane < 4
        wb = jnp.where(first_half, A, C)
        bw = jnp.where(first_half, C, A)
        stm = stm_ref[...]
        acc = stm * wb + (1.0 - stm) * bw + ftb8_ref[...]
        x = _crelu(acc)
        x = _crelu(jnp.dot(x, l1wT_ref[...],
                           preferred_element_type=jnp.float32) + l1b_ref[...])
        x = _crelu(jnp.dot(x, l2wT_ref[...],
                           preferred_element_type=jnp.float32) + l2b_ref[...])
        out_ref[...] = jnp.dot(x, l3wT_ref[...],
                               preferred_element_type=jnp.float32) + l3b_ref[...]


@functools.partial(jax.jit, static_argnames=("bm", "fc"))
def _nnue(wfts, bfts, stm, ft_w, ft_b, l1_w, l1_b, l2_w, l2_b, l3_w, l3_b,
          bm=256, fc=8192):
    B, F = wfts.shape
    ftwT = ft_w.T
    w8 = jnp.concatenate([ftwT, ftwT], axis=1)
    ftb8 = jnp.concatenate([ft_b, ft_b]).reshape(1, 8)
    grid = (B // bm, F // fc)
    return pl.pallas_call(
        _nnue_body,
        grid=grid,
        in_specs=[
            pl.BlockSpec((bm, fc), lambda i, j: (i, j)),
            pl.BlockSpec((bm, fc), lambda i, j: (i, j)),
            pl.BlockSpec((fc, 8), lambda i, j: (j, 0)),
            pl.BlockSpec((bm, 1), lambda i, j: (i, 0)),
            pl.BlockSpec((1, 8), lambda i, j: (0, 0)),
            pl.BlockSpec((8, 8), lambda i, j: (0, 0)),
            pl.BlockSpec((1, 8), lambda i, j: (0, 0)),
            pl.BlockSpec((8, 8), lambda i, j: (0, 0)),
            pl.BlockSpec((1, 8), lambda i, j: (0, 0)),
            pl.BlockSpec((8, 1), lambda i, j: (0, 0)),
            pl.BlockSpec((1, 1), lambda i, j: (0, 0)),
        ],
        out_specs=pl.BlockSpec((bm, 1), lambda i, j: (i, 0)),
        out_shape=jax.ShapeDtypeStruct((B, 1), jnp.float32),
        scratch_shapes=[
            pltpu.VMEM((bm, 8), jnp.float32),
            pltpu.VMEM((bm, 8), jnp.float32),
        ],
        compiler_params=pltpu.CompilerParams(
            dimension_semantics=("parallel", "arbitrary"),
        ),
    )(wfts, bfts, w8, stm, ftb8,
      l1_w.T, l1_b.reshape(1, 8),
      l2_w.T, l2_b.reshape(1, 8),
      l3_w.T, l3_b.reshape(1, 1))


def kernel(wfts, bfts, stm, ft_w, ft_b, l1_w, l1_b, l2_w, l2_b, l3_w, l3_b):
    return _nnue(wfts, bfts, stm, ft_w, ft_b,
                 l1_w, l1_b, l2_w, l2_b, l3_w, l3_b)


# speedup vs baseline: 1.1607x; 1.1607x over previous
import functools

import jax
import jax.numpy as jnp
from jax.experimental import pallas as pl
from jax.experimental.pallas import tpu as pltpu


def _crelu(x):
    return jnp.clip(x, 0.0, 1.0)


def _make_body(S):
    def _nnue_body(*refs):
        (wf_refs, bf_refs, w8_refs, rest) = (
            refs[0:S], refs[S:2 * S], refs[2 * S:3 * S], refs[3 * S:])
        (stm_ref, ftb8_ref, l1wT_ref, l1b_ref, l2wT_ref, l2b_ref,
         l3wT_ref, l3b_ref, out_ref, accA_ref, accC_ref) = rest
        j = pl.program_id(0)

        pA = jnp.dot(wf_refs[0][...], w8_refs[0][...],
                     preferred_element_type=jnp.float32)
        pC = jnp.dot(bf_refs[0][...], w8_refs[0][...],
                     preferred_element_type=jnp.float32)
        for s in range(1, S):
            w8s = w8_refs[s][...]
            pA += jnp.dot(wf_refs[s][...], w8s,
                          preferred_element_type=jnp.float32)
            pC += jnp.dot(bf_refs[s][...], w8s,
                          preferred_element_type=jnp.float32)

        @pl.when(j == 0)
        def _init():
            accA_ref[...] = pA
            accC_ref[...] = pC

        @pl.when(j > 0)
        def _acc():
            accA_ref[...] += pA
            accC_ref[...] += pC

        @pl.when(j == pl.num_programs(0) - 1)
        def _tail():
            A = accA_ref[...]
            C = accC_ref[...]
            lane = jax.lax.broadcasted_iota(jnp.int32, A.shape, 1)
            first_half = lane < 4
            wb = jnp.where(first_half, A, C)
            bw = jnp.where(first_half, C, A)
            stm = stm_ref[...]
            acc = stm * wb + (1.0 - stm) * bw + ftb8_ref[...]
            x = _crelu(acc)
            x = _crelu(jnp.dot(x, l1wT_ref[...],
                               preferred_element_type=jnp.float32)
                       + l1b_ref[...])
            x = _crelu(jnp.dot(x, l2wT_ref[...],
                               preferred_element_type=jnp.float32)
                       + l2b_ref[...])
            out_ref[...] = (jnp.dot(x, l3wT_ref[...],
                                    preferred_element_type=jnp.float32)
                            + l3b_ref[...])
    return _nnue_body


@functools.partial(jax.jit, static_argnames=("fc", "S"))
def _nnue(wfts, bfts, stm, ft_w, ft_b, l1_w, l1_b, l2_w, l2_b, l3_w, l3_b,
          fc=512, S=4):
    B, F = wfts.shape
    ftwT = ft_w.T
    w8 = jnp.concatenate([ftwT, ftwT], axis=1)
    ftb8 = jnp.concatenate([ft_b, ft_b]).reshape(1, 8)
    nsteps = F // (fc * S)

    def data_spec(s):
        return pl.BlockSpec((B, fc), lambda j, s=s: (0, j * S + s))

    def w8_spec(s):
        return pl.BlockSpec((fc, 8), lambda j, s=s: (j * S + s, 0))

    in_specs = ([data_spec(s) for s in range(S)]
                + [data_spec(s) for s in range(S)]
                + [w8_spec(s) for s in range(S)]
                + [
        pl.BlockSpec((B, 1), lambda j: (0, 0)),
        pl.BlockSpec((1, 8), lambda j: (0, 0)),
        pl.BlockSpec((8, 8), lambda j: (0, 0)),
        pl.BlockSpec((1, 8), lambda j: (0, 0)),
        pl.BlockSpec((8, 8), lambda j: (0, 0)),
        pl.BlockSpec((1, 8), lambda j: (0, 0)),
        pl.BlockSpec((8, 1), lambda j: (0, 0)),
        pl.BlockSpec((1, 1), lambda j: (0, 0)),
    ])
    args = ([wfts] * S + [bfts] * S + [w8] * S
            + [stm, ftb8,
               l1_w.T, l1_b.reshape(1, 8),
               l2_w.T, l2_b.reshape(1, 8),
               l3_w.T, l3_b.reshape(1, 1)])
    return pl.pallas_call(
        _make_body(S),
        grid=(nsteps,),
        in_specs=in_specs,
        out_specs=pl.BlockSpec((B, 1), lambda j: (0, 0)),
        out_shape=jax.ShapeDtypeStruct((B, 1), jnp.float32),
        scratch_shapes=[
            pltpu.VMEM((B, 8), jnp.float32),
            pltpu.VMEM((B, 8), jnp.float32),
        ],
        compiler_params=pltpu.CompilerParams(
            dimension_semantics=("arbitrary",),
        ),
    )(*args)


def kernel(wfts, bfts, stm, ft_w, ft_b, l1_w, l1_b, l2_w, l2_b, l3_w, l3_b):
    return _nnue(wfts, bfts, stm, ft_w, ft_b,
                 l1_w, l1_b, l2_w, l2_b, l3_w, l3_b)
